# Initial kernel scaffold; baseline (speedup 1.0000x reference)
import jax
import jax.numpy as jnp
from jax.experimental import pallas as pl


def kernel(idx, tables, W1, b1, g1a, be1a, g1b, be1b, W2, b2, g2a, be2a, g2b, be2b, W3, b3):
    raise NotImplementedError("write your pallas kernel here")


# speedup vs baseline: 2.1193x; 2.1193x over previous
import functools

import jax
import jax.numpy as jnp
from jax import lax
from jax.experimental import pallas as pl
from jax.experimental.pallas import tpu as pltpu
from jax.experimental.pallas import tpu_sc as plsc

_EPS = 1e-5


def _sc_gather(flat_idx, tables2d):
    n = flat_idx.shape[0]
    d = tables2d.shape[1]
    info = plsc.get_sparse_core_info()
    nw = info.num_cores * info.num_subcores
    per_w = n // nw
    ch = 1664
    n_ch = per_w // ch
    assert per_w % ch == 0

    mesh = plsc.VectorSubcoreMesh(core_axis_name="c", subcore_axis_name="s")

    @functools.partial(
        pl.kernel,
        mesh=mesh,
        out_type=jax.ShapeDtypeStruct((n, d), jnp.float32),
        compiler_params=pltpu.CompilerParams(use_tc_tiling_on_sc=False),
        scratch_types=[
            pltpu.VMEM((ch,), jnp.int32),
            pltpu.VMEM((ch, d), jnp.float32),
            pltpu.SemaphoreType.DMA,
        ],
    )
    def gather_kernel(idx_hbm, tab_hbm, out_hbm, idx_v, rows_v, sem):
        wid = lax.axis_index("s") * info.num_cores + lax.axis_index("c")
        base = wid * per_w

        def body(i, carry):
            off = base + i * ch
            pltpu.sync_copy(idx_hbm.at[pl.ds(off, ch)], idx_v)
            pltpu.async_copy(tab_hbm.at[idx_v], rows_v, sem).wait()
            pltpu.sync_copy(rows_v, out_hbm.at[pl.ds(off, ch)])
            return carry

        lax.fori_loop(0, n_ch, body, 0)

    return gather_kernel(flat_idx, tables2d)


def _mm_stats_body(x_ref, w_ref, b_ref, h_ref, s_ref, q_ref):
    j = pl.program_id(0)
    h = jnp.dot(x_ref[...], w_ref[...], preferred_element_type=jnp.float32)
    h = h + b_ref[...]
    h_ref[...] = h

    @pl.when(j == 0)
    def _():
        s_ref[...] = jnp.zeros_like(s_ref)
        q_ref[...] = jnp.zeros_like(q_ref)

    s_ref[...] += jnp.sum(h, axis=0, keepdims=True)
    q_ref[...] += jnp.sum(h * h, axis=0, keepdims=True)


def _bn_affine(s, q, ga, bea, gb, beb, nb):
    m = s / nb
    v = q / nb - m * m
    inv1 = lax.rsqrt(v + _EPS)
    sa = ga * inv1
    v2 = sa * sa * v
    inv2 = lax.rsqrt(v2 + _EPS)
    scale = sa * gb * inv2
    return m, scale


def _norm_mm_stats_body(h_ref, s_in, q_in, ga, bea, gb, beb, w_ref, b_ref,
                        h2_ref, s_ref, q_ref, *, nb):
    j = pl.program_id(0)
    m, scale = _bn_affine(s_in[...], q_in[...], ga[...], bea[...],
                          gb[...], beb[...], nb)
    z = jnp.maximum((h_ref[...] - m) * scale + beb[...], 0.0)
    h2 = jnp.dot(z, w_ref[...], preferred_element_type=jnp.float32)
    h2 = h2 + b_ref[...]
    h2_ref[...] = h2

    @pl.when(j == 0)
    def _():
        s_ref[...] = jnp.zeros_like(s_ref)
        q_ref[...] = jnp.zeros_like(q_ref)

    s_ref[...] += jnp.sum(h2, axis=0, keepdims=True)
    q_ref[...] += jnp.sum(h2 * h2, axis=0, keepdims=True)


def _norm_out_body(h_ref, s_in, q_in, ga, bea, gb, beb, w_ref, b_ref,
                   o_ref, *, nb):
    m, scale = _bn_affine(s_in[...], q_in[...], ga[...], bea[...],
                          gb[...], beb[...], nb)
    z = jnp.maximum((h_ref[...] - m) * scale + beb[...], 0.0)
    o = jnp.dot(z, w_ref[...], preferred_element_type=jnp.float32)
    o_ref[...] = jax.nn.sigmoid(o + b_ref[...])


def _row2(a):
    return a.reshape(1, -1)


def kernel(idx, tables, W1, b1, g1a, be1a, g1b, be1b, W2, b2, g2a, be2a,
           g2b, be2b, W3, b3):
    bsz, g = idx.shape
    _, v, d = tables.shape
    gd, h1d = W1.shape
    h2d = W2.shape[1]

    offs = (jnp.arange(g, dtype=jnp.int32) * v)[None, :]
    flat_idx = (idx.astype(jnp.int32) + offs).reshape(-1)
    rows = _sc_gather(flat_idx, tables.reshape(g * v, d))
    x = rows.reshape(bsz, gd)

    r = 1024
    nblk = bsz // r
    fullspec = lambda shp: pl.BlockSpec(shp, lambda j: (0, 0))

    h1, s1, q1 = pl.pallas_call(
        _mm_stats_body,
        grid=(nblk,),
        in_specs=[
            pl.BlockSpec((r, gd), lambda j: (j, 0)),
            fullspec((gd, h1d)),
            fullspec((1, h1d)),
        ],
        out_specs=[
            pl.BlockSpec((r, h1d), lambda j: (j, 0)),
            fullspec((1, h1d)),
            fullspec((1, h1d)),
        ],
        out_shape=[
            jax.ShapeDtypeStruct((bsz, h1d), jnp.float32),
            jax.ShapeDtypeStruct((1, h1d), jnp.float32),
            jax.ShapeDtypeStruct((1, h1d), jnp.float32),
        ],
    )(x, W1, _row2(b1))

    h2, s2, q2 = pl.pallas_call(
        functools.partial(_norm_mm_stats_body, nb=float(bsz)),
        grid=(nblk,),
        in_specs=[
            pl.BlockSpec((r, h1d), lambda j: (j, 0)),
            fullspec((1, h1d)), fullspec((1, h1d)),
            fullspec((1, h1d)), fullspec((1, h1d)),
            fullspec((1, h1d)), fullspec((1, h1d)),
            fullspec((h1d, h2d)),
            fullspec((1, h2d)),
        ],
        out_specs=[
            pl.BlockSpec((r, h2d), lambda j: (j, 0)),
            fullspec((1, h2d)),
            fullspec((1, h2d)),
        ],
        out_shape=[
            jax.ShapeDtypeStruct((bsz, h2d), jnp.float32),
            jax.ShapeDtypeStruct((1, h2d), jnp.float32),
            jax.ShapeDtypeStruct((1, h2d), jnp.float32),
        ],
    )(h1, s1, q1, _row2(g1a), _row2(be1a), _row2(g1b), _row2(be1b),
      W2, _row2(b2))

    out = pl.pallas_call(
        functools.partial(_norm_out_body, nb=float(bsz)),
        grid=(nblk,),
        in_specs=[
            pl.BlockSpec((r, h2d), lambda j: (j, 0)),
            fullspec((1, h2d)), fullspec((1, h2d)),
            fullspec((1, h2d)), fullspec((1, h2d)),
            fullspec((1, h2d)), fullspec((1, h2d)),
            fullspec((h2d, 1)),
            fullspec((1, 1)),
        ],
        out_specs=pl.BlockSpec((r, 1), lambda j: (j, 0)),
        out_shape=jax.ShapeDtypeStruct((bsz, 1), jnp.float32),
    )(h2, s2, q2, _row2(g2a), _row2(be2a), _row2(g2b), _row2(be2b),
      W3, _row2(b3))

    return out
